# baseline (speedup 1.0000x reference)
import functools

import jax
import jax.numpy as jnp
from jax import lax
from jax.experimental import pallas as pl
from jax.experimental.pallas import tpu as pltpu
from jax.experimental.pallas import tpu_sc as plsc

_PATCH = 16
_CONST = 0.7
_C, _H, _W = 3, 512, 512
_NH, _NW = _H // _PATCH, _W // _PATCH
_LANES = 16


def _body(n_idx, img_hbm, idx_hbm, out_hbm, idx_v, buf,
          li0, li1, li2, so0, so1, so2):
    ci = lax.axis_index("c")
    si = lax.axis_index("s")
    wid = si * 2 + ci
    rows = pl.ds(wid * _PATCH, _PATCH)
    isems = (li0, li1, li2)
    osems = (so0, so1, so2)

    loads = [
        pltpu.async_copy(img_hbm.at[c, rows, :], buf.at[c], isems[c])
        for c in range(_C)
    ]
    pltpu.sync_copy(idx_hbm, idx_v.at[pl.ds(0, n_idx)])

    nk = (n_idx + _LANES - 1) // _LANES
    iota = lax.iota(jnp.int32, _LANES)

    def bit_step(k, acc):
        p = idx_v[pl.ds(k * _LANES, _LANES)]
        valid = (iota + k * _LANES) < n_idx
        m = jnp.logical_and(valid, jnp.right_shift(p, 5) == wid)
        pw = jnp.bitwise_and(p, _NW - 1)
        return jnp.bitwise_or(acc, jnp.where(m, jnp.left_shift(1, pw), 0))

    acc = lax.fori_loop(0, nk, bit_step, jnp.zeros((_LANES,), jnp.int32))
    lanes = [acc[l] for l in range(_LANES)]
    while len(lanes) > 1:
        lanes = [
            jnp.bitwise_or(lanes[i], lanes[i + 1]) if i + 1 < len(lanes)
            else lanes[i]
            for i in range(0, len(lanes), 2)
        ]
    bits = lanes[0]

    cvec = jnp.full((_LANES,), _CONST, jnp.float32)
    stores = []
    for c in range(_C):
        loads[c].wait()

        def col_step(j, carry, c=c):
            @pl.when(jnp.bitwise_and(jnp.right_shift(bits, j), 1) != 0)
            def _erase():
                def row_step(r, rc):
                    buf[c, r, pl.ds(j * _PATCH, _PATCH)] = cvec
                    return rc
                lax.fori_loop(0, _PATCH, row_step, 0)
            return carry

        lax.fori_loop(0, _NW, col_step, 0)
        stores.append(
            pltpu.async_copy(buf.at[c], out_hbm.at[c, rows, :], osems[c])
        )
    for st in stores:
        st.wait()


def kernel(img, erase_indices):
    n_idx = erase_indices.shape[0]
    n_pad = ((n_idx + _LANES - 1) // _LANES) * _LANES
    mesh = plsc.VectorSubcoreMesh(
        core_axis_name="c", subcore_axis_name="s", num_cores=2, num_subcores=16
    )
    run = functools.partial(
        pl.kernel,
        out_type=jax.ShapeDtypeStruct((_C, _H, _W), jnp.float32),
        mesh=mesh,
        scratch_types=[
            pltpu.VMEM((n_pad,), jnp.int32),
            pltpu.VMEM((_C, _PATCH, _W), jnp.float32),
            pltpu.SemaphoreType.DMA,
            pltpu.SemaphoreType.DMA,
            pltpu.SemaphoreType.DMA,
            pltpu.SemaphoreType.DMA,
            pltpu.SemaphoreType.DMA,
            pltpu.SemaphoreType.DMA,
        ],
    )(functools.partial(_body, n_idx))
    return run(img, erase_indices.astype(jnp.int32))


# speedup vs baseline: 1.0642x; 1.0642x over previous
import functools

import jax
import jax.numpy as jnp
from jax import lax
from jax.experimental import pallas as pl
from jax.experimental.pallas import tpu as pltpu
from jax.experimental.pallas import tpu_sc as plsc

_PATCH = 16
_CONST = 0.7
_C, _H, _W = 3, 512, 512
_NH, _NW = _H // _PATCH, _W // _PATCH
_LANES = 16


def _body(n_idx, img_hbm, idx_hbm, out_hbm, idx_v, buf, isem, osem, isem2, osem2):
    ci = lax.axis_index("c")
    si = lax.axis_index("s")
    wid = si * 2 + ci
    rows = pl.ds(wid * _PATCH, _PATCH)

    half0 = pl.ds(wid * _PATCH, _PATCH // 2)
    half1 = pl.ds(wid * _PATCH + _PATCH // 2, _PATCH // 2)
    load0 = pltpu.async_copy(img_hbm.at[:, half0, :], buf.at[:, pl.ds(0, 8)], isem)
    load1 = pltpu.async_copy(img_hbm.at[:, half1, :], buf.at[:, pl.ds(8, 8)], isem2)
    pltpu.sync_copy(idx_hbm, idx_v.at[pl.ds(0, n_idx)])

    nk = (n_idx + _LANES - 1) // _LANES
    iota = lax.iota(jnp.int32, _LANES)

    def bit_step(k, acc):
        p = idx_v[pl.ds(k * _LANES, _LANES)]
        valid = (iota + k * _LANES) < n_idx
        m = jnp.logical_and(valid, jnp.right_shift(p, 5) == wid)
        pw = jnp.bitwise_and(p, _NW - 1)
        return jnp.bitwise_or(acc, jnp.where(m, jnp.left_shift(1, pw), 0))

    acc = lax.fori_loop(0, nk, bit_step, jnp.zeros((_LANES,), jnp.int32))
    lanes = [acc[l] for l in range(_LANES)]
    while len(lanes) > 1:
        lanes = [
            jnp.bitwise_or(lanes[i], lanes[i + 1]) if i + 1 < len(lanes)
            else lanes[i]
            for i in range(0, len(lanes), 2)
        ]
    bits = lanes[0]

    cvec = jnp.full((_LANES,), _CONST, jnp.float32)

    def make_col_step(r0):
        def col_step(j, carry):
            @pl.when(jnp.bitwise_and(jnp.right_shift(bits, j), 1) != 0)
            def _erase():
                def row_step(r, rc):
                    for c in range(_C):
                        buf[c, r0 + r, pl.ds(j * _PATCH, _PATCH)] = cvec
                    return rc
                lax.fori_loop(0, _PATCH // 2, row_step, 0)
            return carry
        return col_step

    load0.wait()
    lax.fori_loop(0, _NW, make_col_step(0), 0)
    st0 = pltpu.async_copy(buf.at[:, pl.ds(0, 8)], out_hbm.at[:, half0, :], osem)
    load1.wait()
    lax.fori_loop(0, _NW, make_col_step(8), 0)
    st1 = pltpu.async_copy(buf.at[:, pl.ds(8, 8)], out_hbm.at[:, half1, :], osem2)
    st0.wait()
    st1.wait()


def kernel(img, erase_indices):
    n_idx = erase_indices.shape[0]
    n_pad = ((n_idx + _LANES - 1) // _LANES) * _LANES
    mesh = plsc.VectorSubcoreMesh(
        core_axis_name="c", subcore_axis_name="s", num_cores=2, num_subcores=16
    )
    run = functools.partial(
        pl.kernel,
        out_type=jax.ShapeDtypeStruct((_C, _H, _W), jnp.float32),
        mesh=mesh,
        scratch_types=[
            pltpu.VMEM((n_pad,), jnp.int32),
            pltpu.VMEM((_C, _PATCH, _W), jnp.float32),
            pltpu.SemaphoreType.DMA,
            pltpu.SemaphoreType.DMA,
            pltpu.SemaphoreType.DMA,
            pltpu.SemaphoreType.DMA,
        ],
    )(functools.partial(_body, n_idx))
    return run(img, erase_indices.astype(jnp.int32))


# speedup vs baseline: 1.0653x; 1.0010x over previous
import functools

import jax
import jax.numpy as jnp
from jax import lax
from jax.experimental import pallas as pl
from jax.experimental.pallas import tpu as pltpu
from jax.experimental.pallas import tpu_sc as plsc

_PATCH = 16
_CONST = 0.7
_C, _H, _W = 3, 512, 512
_NH, _NW = _H // _PATCH, _W // _PATCH
_LANES = 16


def _body(n_idx, img_hbm, idx_hbm, out_hbm, idx_v, buf, isem, osem, isem2, osem2):
    ci = lax.axis_index("c")
    si = lax.axis_index("s")
    wid = si * 2 + ci

    half0 = pl.ds(wid * _PATCH, _PATCH // 2)
    half1 = pl.ds(wid * _PATCH + _PATCH // 2, _PATCH // 2)
    load0 = pltpu.async_copy(img_hbm.at[:, half0, :], buf.at[:, pl.ds(0, 8)], isem)
    load1 = pltpu.async_copy(img_hbm.at[:, half1, :], buf.at[:, pl.ds(8, 8)], isem2)
    pltpu.sync_copy(idx_hbm, idx_v.at[pl.ds(0, n_idx)])

    nk = (n_idx + _LANES - 1) // _LANES
    iota = lax.iota(jnp.int32, _LANES)

    def bit_step(k, acc):
        p = idx_v[pl.ds(k * _LANES, _LANES)]
        valid = (iota + k * _LANES) < n_idx
        m = jnp.logical_and(valid, jnp.right_shift(p, 5) == wid)
        pw = jnp.bitwise_and(p, _NW - 1)
        return jnp.bitwise_or(acc, jnp.where(m, jnp.left_shift(1, pw), 0))

    acc = lax.fori_loop(0, nk, bit_step, jnp.zeros((_LANES,), jnp.int32))
    lanes = [acc[l] for l in range(_LANES)]
    while len(lanes) > 1:
        lanes = [
            jnp.bitwise_or(lanes[i], lanes[i + 1]) if i + 1 < len(lanes)
            else lanes[i]
            for i in range(0, len(lanes), 2)
        ]
    bits = lanes[0]

    cvec = jnp.full((_LANES,), _CONST, jnp.float32)

    def make_col_step(r0):
        def col_step(j, carry):
            @pl.when(jnp.bitwise_and(jnp.right_shift(bits, j), 1) != 0)
            def _erase():
                def row_step(r, rc):
                    for c in range(_C):
                        buf[c, r0 + r, pl.ds(j * _PATCH, _PATCH)] = cvec
                    return rc
                lax.fori_loop(0, _PATCH // 2, row_step, 0)
            return carry
        return col_step

    load0.wait()
    lax.fori_loop(0, _NW, make_col_step(0), 0)
    st0 = pltpu.async_copy(buf.at[:, pl.ds(0, 8)], out_hbm.at[:, half0, :], osem)
    load1.wait()
    lax.fori_loop(0, _NW, make_col_step(8), 0)
    st1 = pltpu.async_copy(buf.at[:, pl.ds(8, 8)], out_hbm.at[:, half1, :], osem2)
    st0.wait()
    st1.wait()


def kernel(img, erase_indices):
    n_idx = erase_indices.shape[0]
    n_pad = ((n_idx + _LANES - 1) // _LANES) * _LANES
    mesh = plsc.VectorSubcoreMesh(
        core_axis_name="c", subcore_axis_name="s", num_cores=2, num_subcores=16
    )
    run = functools.partial(
        pl.kernel,
        out_type=jax.ShapeDtypeStruct((_C, _H, _W), jnp.float32),
        mesh=mesh,
        scratch_types=[
            pltpu.VMEM((n_pad,), jnp.int32),
            pltpu.VMEM((_C, _PATCH, _W), jnp.float32),
            pltpu.SemaphoreType.DMA,
            pltpu.SemaphoreType.DMA,
            pltpu.SemaphoreType.DMA,
            pltpu.SemaphoreType.DMA,
        ],
    )(functools.partial(_body, n_idx))
    return run(img, erase_indices.astype(jnp.int32))
